# baseline (device time: 145317 ns/iter reference)
import jax
import jax.numpy as jnp
from jax import lax
from jax.experimental import pallas as pl
from jax.experimental.pallas import tpu as pltpu

N_DEV = 4
N_BLK = 2048
H = 2


def kernel(x, w_mat):
    k_total, k_per = x.shape
    _, n_total = w_mat.shape
    m_per = k_total // N_DEV
    JH = n_total // (H * N_BLK)

    my_i = lax.axis_index("i").astype(jnp.int32)
    order = jnp.stack([
        jnp.mod(my_i + jnp.array([0, 1, 3, 2], jnp.int32), N_DEV),
        jnp.array([0, 0, 2, 1], jnp.int32),
    ])

    def body(order_ref, x_ref, w_ref, out_ref, acc_ref, comm_ref, chunk_ref,
             send_ref, send_sems, recv_sems, local_sem):

        def stage_block(src_block, sendbuf_slot):
            cp = pltpu.make_async_copy(src_block, chunk_ref, local_sem)
            cp.start()
            cp.wait()
            send_ref[sendbuf_slot] = chunk_ref[:, :].astype(jnp.bfloat16)

        def load_local():
            cp = pltpu.make_async_copy(
                x_ref.at[pl.ds(i * m_per, m_per)], chunk_ref, local_sem
            )
            cp.start()
            cp.wait()

        h = pl.program_id(0)
        p = pl.program_id(1)
        j = pl.program_id(2)
        i = order_ref[0, 0]
        slot = order_ref[1, p]
        first = jnp.logical_and(h == 0, j == 0)

        @pl.when(jnp.logical_and(first, p == 0))
        def _start():
            barrier_sem = pltpu.get_barrier_semaphore()
            for d in (1, 2, 3):
                peer = jnp.mod(i + d, N_DEV)
                pl.semaphore_signal(
                    barrier_sem, inc=1,
                    device_id=(peer,), device_id_type=pl.DeviceIdType.MESH,
                )
            pl.semaphore_wait(barrier_sem, N_DEV - 1)
            for d in (1, 3):
                tgt = jnp.mod(i + d, N_DEV)
                stage_block(x_ref.at[pl.ds(tgt * m_per, m_per)], d - 1)
                pltpu.make_async_remote_copy(
                    src_ref=send_ref.at[d - 1],
                    dst_ref=comm_ref.at[3 - d],
                    send_sem=send_sems.at[d - 1],
                    recv_sem=recv_sems.at[3 - d],
                    device_id=(tgt,),
                    device_id_type=pl.DeviceIdType.MESH,
                ).start()
            load_local()

        @pl.when(jnp.logical_and(first, p == 1))
        def _send_diag():
            tgt = jnp.mod(i + 2, N_DEV)
            stage_block(x_ref.at[pl.ds(tgt * m_per, m_per)], 1)
            pltpu.make_async_remote_copy(
                src_ref=send_ref.at[1],
                dst_ref=comm_ref.at[1],
                send_sem=send_sems.at[1],
                recv_sem=recv_sems.at[1],
                device_id=(tgt,),
                device_id_type=pl.DeviceIdType.MESH,
            ).start()

        @pl.when(jnp.logical_and(first, p > 0))
        def _wait_chunk():
            pltpu.make_async_remote_copy(
                src_ref=send_ref.at[0],
                dst_ref=comm_ref.at[slot],
                send_sem=send_sems.at[0],
                recv_sem=recv_sems.at[slot],
                device_id=(i,),
                device_id_type=pl.DeviceIdType.MESH,
            ).wait_recv()

        @pl.when(jnp.logical_and(j == 0, p > 0))
        def _hoist_chunk():
            chunk_ref[:, :] = comm_ref[slot].astype(jnp.float32)

        @pl.when(jnp.logical_and(jnp.logical_and(h == 1, j == 0), p == 0))
        def _reload_local():
            load_local()

        jsl = pl.ds(j * N_BLK, N_BLK)
        val = jnp.dot(
            chunk_ref[:, :], w_ref[:, :], preferred_element_type=jnp.float32
        )

        @pl.when(p == 0)
        def _init():
            acc_ref[:, jsl] = val

        @pl.when(jnp.logical_and(p > 0, p < N_DEV - 1))
        def _acc():
            acc_ref[:, jsl] += val

        @pl.when(p == N_DEV - 1)
        def _fin():
            t = acc_ref[:, jsl] + val
            out_ref[:, :] = (t * jax.nn.sigmoid(t)).astype(jnp.bfloat16)

        @pl.when(jnp.logical_and(jnp.logical_and(h == H - 1, j == JH - 1),
                                 p == N_DEV - 1))
        def _drain_sends():
            for d in (1, 2, 3):
                tgt = jnp.mod(i + d, N_DEV)
                pltpu.make_async_remote_copy(
                    src_ref=send_ref.at[d - 1],
                    dst_ref=comm_ref.at[3 - d],
                    send_sem=send_sems.at[d - 1],
                    recv_sem=recv_sems.at[3 - d],
                    device_id=(tgt,),
                    device_id_type=pl.DeviceIdType.MESH,
                ).wait_send()

    grid_spec = pltpu.PrefetchScalarGridSpec(
        num_scalar_prefetch=1,
        grid=(H, N_DEV, JH),
        in_specs=[
            pl.BlockSpec(memory_space=pl.ANY),
            pl.BlockSpec(
                (k_per, N_BLK), lambda h, p, j, o: (o[0, p], h * JH + j)
            ),
        ],
        out_specs=pl.BlockSpec(
            (m_per, N_BLK),
            lambda h, p, j, o: (
                0, jnp.where(p == N_DEV - 1, h * JH + j, h * JH)
            ),
        ),
        scratch_shapes=[
            pltpu.VMEM((m_per, JH * N_BLK), jnp.float32),
            pltpu.VMEM((3, m_per, k_per), jnp.bfloat16),
            pltpu.VMEM((m_per, k_per), jnp.float32),
            pltpu.VMEM((3, m_per, k_per), jnp.bfloat16),
            pltpu.SemaphoreType.DMA((3,)),
            pltpu.SemaphoreType.DMA((3,)),
            pltpu.SemaphoreType.DMA,
        ],
    )
    return pl.pallas_call(
        body,
        grid_spec=grid_spec,
        out_shape=jax.ShapeDtypeStruct((m_per, n_total), jnp.bfloat16),
        compiler_params=pltpu.CompilerParams(
            collective_id=0,
            dimension_semantics=("arbitrary", "arbitrary", "arbitrary"),
            vmem_limit_bytes=64 * 1024 * 1024,
        ),
    )(order, x, w_mat)


# device time: 131540 ns/iter; 1.1047x vs baseline; 1.1047x over previous
import jax
import jax.numpy as jnp
from jax import lax
from jax.experimental import pallas as pl
from jax.experimental.pallas import tpu as pltpu

N_DEV = 4
N_BLK = 1024


def kernel(x, w_mat):
    k_total, k_per = x.shape
    _, n_total = w_mat.shape
    m_per = k_total // N_DEV
    J = n_total // N_BLK

    my_i = lax.axis_index("i").astype(jnp.int32)
    order = jnp.stack([
        jnp.mod(my_i + jnp.array([0, 1, 3, 2], jnp.int32), N_DEV),
        jnp.array([0, 0, 2, 1], jnp.int32),
    ])

    def body(order_ref, x_ref, w_ref, out_ref, acc_ref, comm_ref, chunk_ref,
             send_ref, send_sems, recv_sems, local_sem):

        def stage_block(src_block, sendbuf_slot):
            cp = pltpu.make_async_copy(src_block, chunk_ref, local_sem)
            cp.start()
            cp.wait()
            send_ref[sendbuf_slot] = chunk_ref[:, :].astype(jnp.bfloat16)
        p = pl.program_id(0)
        j = pl.program_id(1)
        i = order_ref[0, 0]
        slot = order_ref[1, p]

        @pl.when(jnp.logical_and(p == 0, j == 0))
        def _start():
            barrier_sem = pltpu.get_barrier_semaphore()
            for d in (1, 2, 3):
                peer = jnp.mod(i + d, N_DEV)
                pl.semaphore_signal(
                    barrier_sem, inc=1,
                    device_id=(peer,), device_id_type=pl.DeviceIdType.MESH,
                )
            pl.semaphore_wait(barrier_sem, N_DEV - 1)
            for d in (1, 3):
                tgt = jnp.mod(i + d, N_DEV)
                stage_block(x_ref.at[pl.ds(tgt * m_per, m_per)], d - 1)
                pltpu.make_async_remote_copy(
                    src_ref=send_ref.at[d - 1],
                    dst_ref=comm_ref.at[3 - d],
                    send_sem=send_sems.at[d - 1],
                    recv_sem=recv_sems.at[3 - d],
                    device_id=(tgt,),
                    device_id_type=pl.DeviceIdType.MESH,
                ).start()
            cp = pltpu.make_async_copy(
                x_ref.at[pl.ds(i * m_per, m_per)], chunk_ref, local_sem
            )
            cp.start()
            cp.wait()

        @pl.when(jnp.logical_and(p == 1, j == 0))
        def _send_diag():
            tgt = jnp.mod(i + 2, N_DEV)
            stage_block(x_ref.at[pl.ds(tgt * m_per, m_per)], 1)
            pltpu.make_async_remote_copy(
                src_ref=send_ref.at[1],
                dst_ref=comm_ref.at[1],
                send_sem=send_sems.at[1],
                recv_sem=recv_sems.at[1],
                device_id=(tgt,),
                device_id_type=pl.DeviceIdType.MESH,
            ).start()

        @pl.when(jnp.logical_and(p > 0, j == 0))
        def _wait_chunk():
            pltpu.make_async_remote_copy(
                src_ref=send_ref.at[0],
                dst_ref=comm_ref.at[slot],
                send_sem=send_sems.at[0],
                recv_sem=recv_sems.at[slot],
                device_id=(i,),
                device_id_type=pl.DeviceIdType.MESH,
            ).wait_recv()
            chunk_ref[:, :] = comm_ref[slot].astype(jnp.float32)

        jsl = pl.ds(j * N_BLK, N_BLK)
        val = jnp.dot(
            chunk_ref[:, :], w_ref[:, :], preferred_element_type=jnp.float32
        )

        @pl.when(p == 0)
        def _init():
            acc_ref[:, jsl] = val

        @pl.when(jnp.logical_and(p > 0, p < N_DEV - 1))
        def _acc():
            acc_ref[:, jsl] += val

        @pl.when(p == N_DEV - 1)
        def _fin():
            t = acc_ref[:, jsl] + val
            out_ref[:, :] = (t * jax.nn.sigmoid(t)).astype(jnp.bfloat16)

        @pl.when(jnp.logical_and(p == N_DEV - 1, j == J - 1))
        def _drain_sends():
            for d in (1, 2, 3):
                tgt = jnp.mod(i + d, N_DEV)
                pltpu.make_async_remote_copy(
                    src_ref=send_ref.at[d - 1],
                    dst_ref=comm_ref.at[3 - d],
                    send_sem=send_sems.at[d - 1],
                    recv_sem=recv_sems.at[3 - d],
                    device_id=(tgt,),
                    device_id_type=pl.DeviceIdType.MESH,
                ).wait_send()

    grid_spec = pltpu.PrefetchScalarGridSpec(
        num_scalar_prefetch=1,
        grid=(N_DEV, J),
        in_specs=[
            pl.BlockSpec(memory_space=pl.ANY),
            pl.BlockSpec((k_per, N_BLK), lambda p, j, o: (o[0, p], j)),
        ],
        out_specs=pl.BlockSpec(
            (m_per, N_BLK),
            lambda p, j, o: (0, jnp.where(p == N_DEV - 1, j, 0)),
        ),
        scratch_shapes=[
            pltpu.VMEM((m_per, n_total), jnp.float32),
            pltpu.VMEM((3, m_per, k_per), jnp.bfloat16),
            pltpu.VMEM((m_per, k_per), jnp.float32),
            pltpu.VMEM((3, m_per, k_per), jnp.bfloat16),
            pltpu.SemaphoreType.DMA((3,)),
            pltpu.SemaphoreType.DMA((3,)),
            pltpu.SemaphoreType.DMA,
        ],
    )
    return pl.pallas_call(
        body,
        grid_spec=grid_spec,
        out_shape=jax.ShapeDtypeStruct((m_per, n_total), jnp.bfloat16),
        compiler_params=pltpu.CompilerParams(
            collective_id=0,
            dimension_semantics=("arbitrary", "arbitrary"),
            vmem_limit_bytes=64 * 1024 * 1024,
        ),
    )(order, x, w_mat)
